# baseline (speedup 1.0000x reference)
import jax
import jax.numpy as jnp
from jax import lax
from jax.experimental import pallas as pl
from jax.experimental.pallas import tpu as pltpu
from jax.experimental.pallas import tpu_sc as plsc

EMBED = 64
SRC_LEN = 200
Q_USED = 36
BATCH = 1024
SEQ = Q_USED + SRC_LEN
TOTAL_ROWS = BATCH * SEQ
NC, NS = 2, 16
NW = NC * NS
BPW = BATCH // NW
ROWS_PW = TOTAL_ROWS // NW
BCHUNK = 472
NBCH = ROWS_PW // BCHUNK
NVREG = (SEQ + 15) // 16


def _gather_body(story_hbm, question_hbm, table_hbm, out_hbm,
                 qv, sv, idx_v, buf0, buf1, sem_g0, sem_g1, sem_w0, sem_w1):
    wid = lax.axis_index("s") * NC + lax.axis_index("c")
    b0 = wid * BPW
    r0 = wid * ROWS_PW

    pltpu.sync_copy(question_hbm.at[pl.ds(0, Q_USED), pl.ds(b0, BPW)], qv)
    pltpu.sync_copy(story_hbm.at[:, pl.ds(b0, BPW)], sv)

    lane = lax.iota(jnp.int32, 16)

    def tr_body(b, carry):
        b_vec = jnp.full((16,), b, jnp.int32)
        base = b * SEQ
        for k in range(NVREG):
            t_vec = lane + (16 * k)
            if 16 * (k + 1) <= Q_USED:
                x = plsc.load_gather(qv, [t_vec, b_vec])
                plsc.store_scatter(idx_v, [base + t_vec], x)
            elif 16 * k >= Q_USED and 16 * (k + 1) <= SEQ:
                x = plsc.load_gather(sv, [t_vec - Q_USED, b_vec])
                plsc.store_scatter(idx_v, [base + t_vec], x)
            elif 16 * k < Q_USED:
                mq = t_vec < Q_USED
                xq = plsc.load_gather(qv, [jnp.where(mq, t_vec, 0), b_vec],
                                      mask=mq)
                xs = plsc.load_gather(
                    sv, [jnp.where(mq, 0, t_vec - Q_USED), b_vec], mask=~mq)
                plsc.store_scatter(idx_v, [base + t_vec],
                                   jnp.where(mq, xq, xs))
            else:
                mt = t_vec < SEQ
                x = plsc.load_gather(
                    sv, [jnp.where(mt, t_vec - Q_USED, 0), b_vec], mask=mt)
                plsc.store_scatter(idx_v, [base + t_vec], x, mask=mt)
        return carry

    lax.fori_loop(0, BPW, tr_body, 0)

    bufs = (buf0, buf1)
    sem_g = (sem_g0, sem_g1)
    sem_w = (sem_w0, sem_w1)
    gd = [None] * NBCH
    wd = [None] * NBCH

    def start_gather(g):
        gd[g] = pltpu.async_copy(
            table_hbm.at[idx_v.at[pl.ds(g * BCHUNK, BCHUNK)]],
            bufs[g % 2], sem_g[g % 2])

    start_gather(0)
    for g in range(NBCH):
        slot = g % 2
        gd[g].wait()
        if g >= 1:
            wd[g - 1].wait()
        if g < NBCH - 1:
            start_gather(g + 1)
        wd[g] = pltpu.async_copy(
            bufs[slot], out_hbm.at[pl.ds(r0 + g * BCHUNK, BCHUNK)],
            sem_w[slot])
    wd[NBCH - 1].wait()


def kernel(story, question, embed_table):
    mesh = plsc.VectorSubcoreMesh(
        core_axis_name="c", subcore_axis_name="s",
        num_cores=NC, num_subcores=NS,
    )
    out = pl.kernel(
        _gather_body,
        out_type=jax.ShapeDtypeStruct((TOTAL_ROWS, EMBED), jnp.float32),
        mesh=mesh,
        scratch_types=[
            pltpu.VMEM((Q_USED, BPW), jnp.int32),
            pltpu.VMEM((SRC_LEN, BPW), jnp.int32),
            pltpu.VMEM((ROWS_PW,), jnp.int32),
            pltpu.VMEM((BCHUNK, EMBED), jnp.float32),
            pltpu.VMEM((BCHUNK, EMBED), jnp.float32),
            pltpu.SemaphoreType.DMA,
            pltpu.SemaphoreType.DMA,
            pltpu.SemaphoreType.DMA,
            pltpu.SemaphoreType.DMA,
        ],
        compiler_params=pltpu.CompilerParams(
            use_tc_tiling_on_sc=False, needs_layout_passes=False),
    )(story.astype(jnp.int32), question.astype(jnp.int32), embed_table)
    return out.reshape(BATCH, SEQ, EMBED)
